# Initial kernel scaffold; baseline (speedup 1.0000x reference)
import jax
import jax.numpy as jnp
from jax.experimental import pallas as pl


def kernel(xyz, center):
    raise NotImplementedError("write your pallas kernel here")


# speedup vs baseline: 2.2228x; 2.2228x over previous
import functools

import jax
import jax.numpy as jnp
from jax.experimental import pallas as pl
from jax.experimental.pallas import tpu as pltpu

_B, _N, _G, _K = 16, 8192, 256, 32
_CH = 1024
_NCH = _N // _CH
_BIG = 1 << 30


def _tc_body(x_ref, c_ref, idx_ref, nbr_ref, dist_ref):
    b = pl.program_id(0)
    cmat = c_ref[0]

    def dist_chunk(c, carry):
        xs = x_ref[0, pl.ds(c * _CH, _CH), :]
        d0 = cmat[0:1, :] - xs[:, 0:1]
        d1 = cmat[1:2, :] - xs[:, 1:2]
        d2 = cmat[2:3, :] - xs[:, 2:3]
        dist_ref[pl.ds(c * _CH, _CH), :] = (d0 * d0 + d1 * d1) + d2 * d2
        return carry

    jax.lax.fori_loop(0, _NCH, dist_chunk, 0)

    row32 = jax.lax.broadcasted_iota(jnp.int32, (_K, _G), 0)
    row96 = jax.lax.broadcasted_iota(jnp.int32, (3 * _K, _G), 0)
    rowdiv = row96 // 3
    rowmod = row96 - 3 * rowdiv

    def j_body(j, carry):
        m_prev, am_prev = carry

        def scan_a(c, acc):
            m_run, am_run = acc
            vals = dist_ref[pl.ds(c * _CH, _CH), :]
            iota = jax.lax.broadcasted_iota(jnp.int32, (_CH, _G), 0) + c * _CH
            excl = (vals < m_prev) | ((vals == m_prev) & (iota <= am_prev))
            masked = jnp.where(excl, jnp.inf, vals)
            lm = jnp.min(masked, axis=0, keepdims=True)
            lam = jnp.min(jnp.where(masked == lm, iota, _BIG), axis=0,
                          keepdims=True)
            better = lm < m_run
            return (jnp.where(better, lm, m_run),
                    jnp.where(better, lam, am_run))

        m_j, am_j = jax.lax.fori_loop(
            0, _NCH, scan_a,
            (jnp.full((1, _G), jnp.inf, jnp.float32),
             jnp.full((1, _G), _BIG, jnp.int32)))

        def scan_b(c, acc):
            p0, p1, p2 = acc
            xs = x_ref[0, pl.ds(c * _CH, _CH), :]
            iota = jax.lax.broadcasted_iota(jnp.int32, (_CH, _G), 0) + c * _CH
            sel = iota == am_j
            p0 = p0 + jnp.sum(jnp.where(sel, xs[:, 0:1], 0.0), axis=0,
                              keepdims=True)
            p1 = p1 + jnp.sum(jnp.where(sel, xs[:, 1:2], 0.0), axis=0,
                              keepdims=True)
            p2 = p2 + jnp.sum(jnp.where(sel, xs[:, 2:3], 0.0), axis=0,
                              keepdims=True)
            return (p0, p1, p2)

        zero = jnp.zeros((1, _G), jnp.float32)
        p0, p1, p2 = jax.lax.fori_loop(0, _NCH, scan_b, (zero, zero, zero))

        idxv = am_j + b * _N
        idx_ref[0] = jnp.where(row32 == j, jnp.broadcast_to(idxv, (_K, _G)),
                               idx_ref[0])
        n0 = jnp.broadcast_to(p0 - cmat[0:1, :], (3 * _K, _G))
        n1 = jnp.broadcast_to(p1 - cmat[1:2, :], (3 * _K, _G))
        n2 = jnp.broadcast_to(p2 - cmat[2:3, :], (3 * _K, _G))
        val96 = jnp.where(rowmod == 0, n0, jnp.where(rowmod == 1, n1, n2))
        nbr_ref[0] = jnp.where(rowdiv == j, val96, nbr_ref[0])
        return (m_j, am_j)

    jax.lax.fori_loop(
        0, _K, j_body,
        (jnp.full((1, _G), -jnp.inf, jnp.float32),
         jnp.full((1, _G), jnp.int32(-1))))


_PCALL_KW = dict(
    grid=(_B,),
    in_specs=[
        pl.BlockSpec((1, _N, 8), lambda b: (b, 0, 0)),
        pl.BlockSpec((1, 8, _G), lambda b: (b, 0, 0)),
    ],
    out_specs=[
        pl.BlockSpec((1, _K, _G), lambda b: (b, 0, 0)),
        pl.BlockSpec((1, 3 * _K, _G), lambda b: (b, 0, 0)),
    ],
    out_shape=[
        jax.ShapeDtypeStruct((_B, _K, _G), jnp.int32),
        jax.ShapeDtypeStruct((_B, 3 * _K, _G), jnp.float32),
    ],
    scratch_shapes=[pltpu.VMEM((_N, _G), jnp.float32)],
)


@jax.jit
def _run(xp, cp):
    return pl.pallas_call(_tc_body, **_PCALL_KW)(xp, cp)


def kernel(xyz, center):
    xp = jnp.pad(xyz, ((0, 0), (0, 0), (0, 5)))
    cp = jnp.pad(center, ((0, 0), (0, 0), (0, 5))).transpose(0, 2, 1)
    idx_t, nbr_t = _run(xp, cp)
    idx_flat = idx_t.transpose(0, 2, 1).reshape(-1)
    neighborhood = nbr_t.transpose(0, 2, 1).reshape(_B, _G, _K, 3)
    return (neighborhood, center, idx_flat)


# speedup vs baseline: 3.9652x; 1.7839x over previous
import functools

import jax
import jax.numpy as jnp
from jax import lax
from jax.experimental import pallas as pl
from jax.experimental.pallas import tpu as pltpu
from jax.experimental.pallas import tpu_sc as plsc

_B, _N, _G, _K = 16, 8192, 256, 32
_CH = 1024
_NCH = _N // _CH
_BIG = 1 << 30


def _tc_body(x_ref, c_ref, idx_ref, dist_ref):
    b = pl.program_id(0)
    cmat = c_ref[0]

    def dist_chunk(c, carry):
        xs = x_ref[0, pl.ds(c * _CH, _CH), :]
        d0 = cmat[0:1, :] - xs[:, 0:1]
        d1 = cmat[1:2, :] - xs[:, 1:2]
        d2 = cmat[2:3, :] - xs[:, 2:3]
        dist_ref[pl.ds(c * _CH, _CH), :] = (d0 * d0 + d1 * d1) + d2 * d2
        return carry

    jax.lax.fori_loop(0, _NCH, dist_chunk, 0)

    row32 = jax.lax.broadcasted_iota(jnp.int32, (_K, _G), 0)

    def j_body(j, carry):
        m_prev, am_prev = carry

        def scan_a(c, acc):
            m_run, am_run = acc
            vals = dist_ref[pl.ds(c * _CH, _CH), :]
            iota = jax.lax.broadcasted_iota(jnp.int32, (_CH, _G), 0) + c * _CH
            excl = (vals < m_prev) | ((vals == m_prev) & (iota <= am_prev))
            masked = jnp.where(excl, jnp.inf, vals)
            lm = jnp.min(masked, axis=0, keepdims=True)
            lam = jnp.min(jnp.where(masked == lm, iota, _BIG), axis=0,
                          keepdims=True)
            better = lm < m_run
            return (jnp.where(better, lm, m_run),
                    jnp.where(better, lam, am_run))

        m_j, am_j = jax.lax.fori_loop(
            0, _NCH, scan_a,
            (jnp.full((1, _G), jnp.inf, jnp.float32),
             jnp.full((1, _G), _BIG, jnp.int32)))

        idxv = am_j + b * _N
        idx_ref[0] = jnp.where(row32 == j, jnp.broadcast_to(idxv, (_K, _G)),
                               idx_ref[0])
        return (m_j, am_j)

    jax.lax.fori_loop(
        0, _K, j_body,
        (jnp.full((1, _G), -jnp.inf, jnp.float32),
         jnp.full((1, _G), jnp.int32(-1))))


_PCALL_KW = dict(
    grid=(_B,),
    in_specs=[
        pl.BlockSpec((1, _N, 8), lambda b: (b, 0, 0)),
        pl.BlockSpec((1, 8, _G), lambda b: (b, 0, 0)),
    ],
    out_specs=pl.BlockSpec((1, _K, _G), lambda b: (b, 0, 0)),
    out_shape=jax.ShapeDtypeStruct((_B, _K, _G), jnp.int32),
    scratch_shapes=[pltpu.VMEM((_N, _G), jnp.float32)],
)


_NW = 32
_ROWS = _B * _G * _K
_RPW = (_B * _G) // _NW
_IPW = _RPW * _K
_ICH = 128
_NIC = _IPW // _ICH
_D = 16


def _sc_gather_body(idx_hbm, tab_hbm, cen_hbm, out_hbm, idxv, rows, cents,
                    sem):
    w = lax.axis_index("s") * 2 + lax.axis_index("c")
    pltpu.sync_copy(idx_hbm.at[pl.ds(w * _NIC, _NIC)], idxv)
    pltpu.sync_copy(cen_hbm.at[pl.ds(w * _RPW, _RPW)], cents)

    def gather_chunk(c, carry):
        pltpu.async_copy(tab_hbm.at[idxv.at[c]],
                         rows.at[pl.ds(c * _ICH, _ICH)], sem).wait()
        return carry

    lax.fori_loop(0, _NIC, gather_chunk, 0)

    def sub_group(g, carry):
        cvec = cents[g]
        for i in range(_K):
            rows[g * _K + i] = rows[g * _K + i] - cvec
        return carry

    lax.fori_loop(0, _RPW, sub_group, 0)
    pltpu.sync_copy(rows, out_hbm.at[pl.ds(w * _IPW, _IPW)])


_sc_gather = pl.kernel(
    _sc_gather_body,
    out_type=jax.ShapeDtypeStruct((_ROWS, _D), jnp.float32),
    mesh=plsc.VectorSubcoreMesh(core_axis_name="c", subcore_axis_name="s"),
    compiler_params=pltpu.CompilerParams(use_tc_tiling_on_sc=False),
    scratch_types=[
        pltpu.VMEM((_NIC, _ICH), jnp.int32),
        pltpu.VMEM((_IPW, _D), jnp.float32),
        pltpu.VMEM((_RPW, _D), jnp.float32),
        pltpu.SemaphoreType.DMA,
    ],
)


@jax.jit
def _run(xyz, center):
    xp = jnp.pad(xyz, ((0, 0), (0, 0), (0, 5)))
    cp = jnp.pad(center, ((0, 0), (0, 0), (0, 5))).transpose(0, 2, 1)
    idx_t = pl.pallas_call(_tc_body, **_PCALL_KW)(xp, cp)
    idx_flat = idx_t.transpose(0, 2, 1).reshape(-1)

    tab = jnp.pad(xyz.reshape(_B * _N, 3), ((0, 0), (0, _D - 3)))
    cen = jnp.pad(center.reshape(_B * _G, 3), ((0, 0), (0, _D - 3)))
    idx2d = idx_flat.reshape(_ROWS // _ICH, _ICH)
    out = _sc_gather(idx2d, tab, cen)
    neighborhood = out[:, :3].reshape(_B, _G, _K, 3)
    return neighborhood, idx_flat


def kernel(xyz, center):
    neighborhood, idx_flat = _run(xyz, center)
    return (neighborhood, center, idx_flat)
